# Initial kernel scaffold; baseline (speedup 1.0000x reference)
import jax
import jax.numpy as jnp
from jax.experimental import pallas as pl


def kernel(x):
    raise NotImplementedError("write your pallas kernel here")


# speedup vs baseline: 3.4122x; 3.4122x over previous
import functools

import jax
import jax.numpy as jnp
from jax import lax
from jax.experimental import pallas as pl
from jax.experimental.pallas import tpu as pltpu
from jax.experimental.pallas import tpu_sc as plsc

B = 2048
V = 100
U = 100
THRESH = 0.5
MASKED = 2.0

NC = 2
NS = 16
L = 16
NW = NC * NS
GROUPS = B // (NW * L)
VCHUNK = 20
NCHUNK = V // VCHUNK
CHUNK_COLS = VCHUNK * U


def _tec_body(x_hbm, size_hbm, seq_hbm, xbuf0, xbuf1, thr, seqbuf, sizebuf,
              sem0, sem1):
    wid = lax.axis_index("s") * NC + lax.axis_index("c")
    base_b = wid * (GROUPS * L)

    lane = lax.broadcasted_iota(jnp.int32, (L,), 0)
    zero_f = jnp.zeros((L,), jnp.float32)
    zero_i = jnp.zeros((L,), jnp.int32)
    thresh_v = jnp.full((L,), THRESH, jnp.float32)
    masked_v = jnp.full((L,), MASKED, jnp.float32)

    xbufs = (xbuf0, xbuf1)
    sems = (sem0, sem1)

    def start_copy(flat_idx):
        g, c = divmod(flat_idx, NCHUNK)
        k = flat_idx % 2
        b0 = base_b + g * L
        return pltpu.make_async_copy(
            x_hbm.at[pl.ds(b0, L), pl.ds(c * CHUNK_COLS, CHUNK_COLS)],
            xbufs[k], sems[k])

    cp = start_copy(0)
    cp.start()
    pending = cp

    def merge(ma, ia, mb, ib):
        take_b = (mb > ma) | ((mb == ma) & (ib < ia))
        return jnp.where(take_b, mb, ma), jnp.where(take_b, ib, ia)

    for g in range(GROUPS):
        b0 = base_b + g * L

        def thr_reset(u, _):
            thr[u] = thresh_v
            return 0
        lax.fori_loop(0, U, thr_reset, 0)

        size = zero_f
        for c in range(NCHUNK):
            flat = g * NCHUNK + c
            xbuf = xbufs[flat % 2]
            if flat + 1 < GROUPS * NCHUNK:
                nxt = start_copy(flat + 1)
                nxt.start()
            pending.wait()
            if flat + 1 < GROUPS * NCHUNK:
                pending = nxt

            def v_body(vl, size):
                def u_body(j, carry):
                    col, mx0, am0, mx1, am1, mx2, am2, mx3, am3 = carry
                    accs = [(mx0, am0), (mx1, am1), (mx2, am2), (mx3, am3)]
                    out = []
                    for k in range(4):
                        mx, am = accs[k]
                        w = plsc.load_gather(xbuf, [lane, col])
                        tv = thr[4 * j + k]
                        a = jnp.where(w < tv, zero_f, w)
                        gt = a > mx
                        am = jnp.where(gt, col, am)
                        mx = jnp.maximum(mx, a)
                        out.append((mx, am))
                        col = col + 1
                    (mx0, am0), (mx1, am1), (mx2, am2), (mx3, am3) = out
                    return (col, mx0, am0, mx1, am1, mx2, am2, mx3, am3)

                col0 = jnp.full((L,), vl * U, jnp.int32)
                carry = (col0, zero_f, zero_i, zero_f, zero_i,
                         zero_f, zero_i, zero_f, zero_i)
                carry = lax.fori_loop(0, U // 4, u_body, carry)
                colf, mx0, am0, mx1, am1, mx2, am2, mx3, am3 = carry
                m, am = merge(mx0, am0, mx1, am1)
                m2, am2m = merge(mx2, am2, mx3, am3)
                m, am = merge(m, am, m2, am2m)
                pos = m > zero_f
                sel0 = am - colf + U
                sel0 = jnp.where(pos, sel0, zero_i)
                sel = jnp.where(pos, sel0 + 1, zero_i)
                tcol = jnp.full((L,), c * VCHUNK + vl, jnp.int32)
                plsc.store_scatter(seqbuf, [lane, tcol], sel)
                plsc.store_scatter(thr, [sel0, lane], masked_v, mask=pos)
                return size + m

            size = lax.fori_loop(0, VCHUNK, v_body, size)

        sizebuf[...] = -size
        pltpu.sync_copy(seqbuf, seq_hbm.at[pl.ds(b0, L), :])
        pltpu.sync_copy(sizebuf, size_hbm.at[pl.ds(b0, L)])


@jax.jit
def kernel(x):
    x2d = x.reshape(B, V * U)
    mesh = plsc.VectorSubcoreMesh(core_axis_name="c", subcore_axis_name="s",
                                  num_cores=NC, num_subcores=NS)
    run = pl.kernel(
        _tec_body,
        out_type=(
            jax.ShapeDtypeStruct((B,), jnp.float32),
            jax.ShapeDtypeStruct((B, V), jnp.int32),
        ),
        mesh=mesh,
        scratch_types=[
            pltpu.VMEM((L, CHUNK_COLS), jnp.float32),
            pltpu.VMEM((L, CHUNK_COLS), jnp.float32),
            pltpu.VMEM((U, L), jnp.float32),
            pltpu.VMEM((L, V), jnp.int32),
            pltpu.VMEM((L,), jnp.float32),
            pltpu.SemaphoreType.DMA,
            pltpu.SemaphoreType.DMA,
        ],
        compiler_params=pltpu.CompilerParams(use_tc_tiling_on_sc=False,
                                             needs_layout_passes=False),
        name="greedy_thresh_sc",
    )
    neg_size, seq = run(x2d)
    return (neg_size, seq)
